# baseline (speedup 1.0000x reference)
import functools

import jax
import jax.numpy as jnp
from jax import lax
from jax.experimental import pallas as pl
from jax.experimental.pallas import tpu as pltpu
from jax.experimental.pallas import tpu_sc as plsc

L = 16
CHUNK = 128
RPS = 4
N_WORKERS = 32


def _make_kernel(B, K):
    bpw = B // N_WORKERS
    n_chunks = bpw // CHUNK
    groups = CHUNK // L
    mesh = plsc.VectorSubcoreMesh(core_axis_name="c", subcore_axis_name="s")

    @functools.partial(
        pl.kernel,
        out_type=jax.ShapeDtypeStruct((B,), jnp.float32),
        mesh=mesh,
        compiler_params=pltpu.CompilerParams(
            needs_layout_passes=False, use_tc_tiling_on_sc=True),
        scratch_types=[
            pltpu.VMEM((n_chunks, CHUNK), jnp.int32),
            pltpu.VMEM((n_chunks, CHUNK), jnp.int32),
            pltpu.VMEM((n_chunks, CHUNK), jnp.int32),
            pltpu.VMEM((n_chunks, CHUNK), jnp.int32),
            pltpu.VMEM((2, CHUNK, RPS * K), jnp.float32),
            pltpu.VMEM((2, CHUNK, RPS * K), jnp.float32),
            pltpu.VMEM((bpw,), jnp.float32),
            pltpu.SemaphoreType.DMA,
        ],
    )
    def deep_mf(pu_hbm, qi_hbm, users_hbm, items_hbm, out_hbm,
                uidx_v, iidx_v, usup_v, isup_v, ubuf_v, ibuf_v, out_v, sem):
        wid = lax.axis_index("s") * 2 + lax.axis_index("c")
        chunk_base = wid * n_chunks

        pltpu.sync_copy(users_hbm.at[pl.ds(chunk_base, n_chunks)], uidx_v)
        pltpu.sync_copy(items_hbm.at[pl.ds(chunk_base, n_chunks)], iidx_v)

        for c in range(n_chunks):
            for l in range(CHUNK // L):
                s = pl.ds(l * L, L)
                usup_v[c, s] = lax.shift_right_logical(uidx_v[c, s], 2)
                isup_v[c, s] = lax.shift_right_logical(iidx_v[c, s], 2)

        def start(c):
            buf = c % 2
            return (
                pltpu.async_copy(pu_hbm.at[usup_v.at[c]], ubuf_v.at[buf], sem),
                pltpu.async_copy(qi_hbm.at[isup_v.at[c]], ibuf_v.at[buf], sem),
            )

        lane = lax.iota(jnp.int32, L)
        pending = start(0)

        for c in range(n_chunks):
            for d in pending:
                d.wait()
            if c + 1 < n_chunks:
                nxt = start(c + 1)
            buf = c % 2

            def group_body(g, carry, c=c, buf=buf):
                rows = g * L + lane
                uraw = uidx_v[c, pl.ds(g * L, L)]
                iraw = iidx_v[c, pl.ds(g * L, L)]
                ucol = (uraw & (RPS - 1)) * K
                icol = (iraw & (RPS - 1)) * K
                acc = jnp.zeros((L,), jnp.float32)
                for j in range(K):
                    uj = plsc.load_gather(ubuf_v.at[buf], [rows, ucol + j])
                    vj = plsc.load_gather(ibuf_v.at[buf], [rows, icol + j])
                    acc = acc + uj * vj
                out_v[pl.ds(c * CHUNK + g * L, L)] = acc
                return carry

            lax.fori_loop(0, groups, group_body, 0)
            if c + 1 < n_chunks:
                pending = nxt

        pltpu.sync_copy(out_v, out_hbm.at[pl.ds(wid * bpw, bpw)])

    return deep_mf


@jax.jit
def kernel(users, items, pu_table, qi_table):
    B = users.shape[0]
    n_rows, K = pu_table.shape
    pu2 = pu_table.reshape(n_rows // RPS, RPS * K)
    qi2 = qi_table.reshape(qi_table.shape[0] // RPS, RPS * K)
    users2d = users.reshape(-1).astype(jnp.int32).reshape(-1, CHUNK)
    items2d = items.reshape(-1).astype(jnp.int32).reshape(-1, CHUNK)
    out = _make_kernel(B, K)(pu2, qi2, users2d, items2d)
    return out.reshape(B, 1)


# speedup vs baseline: 1.0004x; 1.0004x over previous
import functools

import jax
import jax.numpy as jnp
from jax import lax
from jax.experimental import pallas as pl
from jax.experimental.pallas import tpu as pltpu
from jax.experimental.pallas import tpu_sc as plsc

L = 16
RPB = 4
CHUNK = 128
N_WORKERS = 32


def _make_kernel(B, K):
    bpw = B // N_WORKERS
    n_chunks = bpw // CHUNK
    groups = CHUNK // L
    blk_w = RPB * K
    mesh = plsc.VectorSubcoreMesh(core_axis_name="c", subcore_axis_name="s")

    @functools.partial(
        pl.kernel,
        out_type=jax.ShapeDtypeStruct((B,), jnp.float32),
        mesh=mesh,
        compiler_params=pltpu.CompilerParams(needs_layout_passes=False),
        scratch_types=[
            pltpu.VMEM((n_chunks, CHUNK), jnp.int32),
            pltpu.VMEM((n_chunks, CHUNK), jnp.int32),
            pltpu.VMEM((n_chunks, CHUNK), jnp.int32),
            pltpu.VMEM((n_chunks, CHUNK), jnp.int32),
            pltpu.VMEM((2, CHUNK, blk_w), jnp.float32),
            pltpu.VMEM((2, CHUNK, blk_w), jnp.float32),
            pltpu.VMEM((bpw,), jnp.float32),
            pltpu.SemaphoreType.DMA,
            pltpu.SemaphoreType.DMA,
        ],
    )
    def deep_mf(pu_hbm, qi_hbm, users_hbm, items_hbm, out_hbm,
                uidx_v, iidx_v, ublk_v, iblk_v, ubuf_v, ibuf_v, out_v,
                usem, isem):
        wid = lax.axis_index("s") * 2 + lax.axis_index("c")

        pltpu.sync_copy(users_hbm.at[wid], uidx_v)
        pltpu.sync_copy(items_hbm.at[wid], iidx_v)

        for c in range(n_chunks):
            for l in range(CHUNK // L):
                s = pl.ds(l * L, L)
                ublk_v[c, s] = lax.shift_right_logical(uidx_v[c, s], 2)
                iblk_v[c, s] = lax.shift_right_logical(iidx_v[c, s], 2)

        def start(c):
            buf = c % 2
            return (
                pltpu.async_copy(pu_hbm.at[ublk_v.at[c]], ubuf_v.at[buf],
                                 usem),
                pltpu.async_copy(qi_hbm.at[iblk_v.at[c]], ibuf_v.at[buf],
                                 isem),
            )

        lane = lax.iota(jnp.int32, L)
        pending = start(0)

        for c in range(n_chunks):
            for d in pending:
                d.wait()
            if c + 1 < n_chunks:
                nxt = start(c + 1)
            buf = c % 2

            def group_body(g, carry, c=c, buf=buf):
                rows = g * L + lane
                s = pl.ds(g * L, L)
                uoff = (uidx_v[c, s] & (RPB - 1)) << 5
                ioff = (iidx_v[c, s] & (RPB - 1)) << 5
                acc = jnp.zeros((L,), jnp.float32)
                for j in range(K):
                    uj = plsc.load_gather(ubuf_v.at[buf], [rows, uoff + j])
                    vj = plsc.load_gather(ibuf_v.at[buf], [rows, ioff + j])
                    acc = acc + uj * vj
                out_v[pl.ds(c * CHUNK + g * L, L)] = acc
                return carry

            lax.fori_loop(0, groups, group_body, 0)
            if c + 1 < n_chunks:
                pending = nxt

        pltpu.sync_copy(out_v, out_hbm.at[pl.ds(wid * bpw, bpw)])

    return deep_mf


@jax.jit
def kernel(users, items, pu_table, qi_table):
    B = users.shape[0]
    K = pu_table.shape[1]
    pu2 = pu_table.reshape(-1, RPB * K)
    qi2 = qi_table.reshape(-1, RPB * K)
    users3d = users.reshape(-1).astype(jnp.int32).reshape(N_WORKERS, -1, CHUNK)
    items3d = items.reshape(-1).astype(jnp.int32).reshape(N_WORKERS, -1, CHUNK)
    out = _make_kernel(B, K)(pu2, qi2, users3d, items3d)
    return out.reshape(B, 1)


# speedup vs baseline: 1.4857x; 1.4852x over previous
import functools

import jax
import jax.numpy as jnp
from jax import lax
from jax.experimental import pallas as pl
from jax.experimental.pallas import tpu as pltpu
from jax.experimental.pallas import tpu_sc as plsc

L = 16
N_WORKERS = 32
C = 128


def _make_kernel(B, K):
    bpw = B // N_WORKERS
    nchunks = bpw // C
    mesh = plsc.VectorSubcoreMesh(core_axis_name="c", subcore_axis_name="s")

    @functools.partial(
        pl.kernel,
        out_type=jax.ShapeDtypeStruct((B,), jnp.float32),
        mesh=mesh,
        compiler_params=pltpu.CompilerParams(needs_layout_passes=False),
        scratch_types=[
            pltpu.VMEM((bpw,), jnp.int32),
            pltpu.VMEM((bpw,), jnp.int32),
            pltpu.VMEM((2, C, K), jnp.float32),
            pltpu.VMEM((2, C, K), jnp.float32),
            pltpu.VMEM((bpw,), jnp.float32),
            pltpu.SemaphoreType.DMA,
            pltpu.SemaphoreType.DMA,
            pltpu.SemaphoreType.DMA,
            pltpu.SemaphoreType.DMA,
        ],
    )
    def deep_mf(pu_hbm, qi_hbm, users_hbm, items_hbm, out_hbm,
                uidx_v, iidx_v, ubuf_v, ibuf_v, out_v,
                usem0, usem1, isem0, isem1):
        wid = lax.axis_index("s") * 2 + lax.axis_index("c")
        usems = (usem0, usem1)
        isems = (isem0, isem1)

        pltpu.sync_copy(users_hbm.at[wid], uidx_v)
        pltpu.sync_copy(items_hbm.at[wid], iidx_v)

        def fire(chunk, slot):
            ub = ubuf_v.at[slot]
            ib = ibuf_v.at[slot]
            usem = usems[slot]
            isem = isems[slot]

            def body(g, carry):
                base = chunk * C + g * L
                uvec = uidx_v[pl.ds(base, L)]
                ivec = iidx_v[pl.ds(base, L)]
                for k in range(L):
                    row = g * L + k
                    pltpu.async_copy(
                        pu_hbm.at[pl.ds(uvec[k], 1)],
                        ub.at[pl.ds(row, 1)], usem)
                    pltpu.async_copy(
                        qi_hbm.at[pl.ds(ivec[k], 1)],
                        ib.at[pl.ds(row, 1)], isem)
                return carry

            lax.fori_loop(0, C // L, body, 0)

        def drain(slot):
            pltpu.make_async_copy(
                pu_hbm.at[pl.ds(0, C)], ubuf_v.at[slot], usems[slot]).wait()
            pltpu.make_async_copy(
                qi_hbm.at[pl.ds(0, C)], ibuf_v.at[slot], isems[slot]).wait()

        lane = lax.iota(jnp.int32, L)

        def reduce_chunk(chunk, slot):
            ub = ubuf_v.at[slot]
            ib = ibuf_v.at[slot]

            def body(g, carry):
                rows = g * L + lane
                acc = jnp.zeros((L,), jnp.float32)
                for j in range(K):
                    col = jnp.full((L,), j, jnp.int32)
                    uj = plsc.load_gather(ub, [rows, col])
                    vj = plsc.load_gather(ib, [rows, col])
                    acc = acc + uj * vj
                out_v[pl.ds(chunk * C + g * L, L)] = acc
                return carry

            lax.fori_loop(0, C // L, body, 0)

        fire(0, 0)
        fire(1, 1)
        for chunk in range(nchunks):
            slot = chunk % 2
            drain(slot)
            reduce_chunk(chunk, slot)
            if chunk + 2 < nchunks:
                fire(chunk + 2, slot)

        pltpu.sync_copy(out_v, out_hbm.at[pl.ds(wid * bpw, bpw)])

    return deep_mf


@jax.jit
def kernel(users, items, pu_table, qi_table):
    B = users.shape[0]
    K = pu_table.shape[1]
    users2d = users.reshape(-1).astype(jnp.int32).reshape(N_WORKERS, -1)
    items2d = items.reshape(-1).astype(jnp.int32).reshape(N_WORKERS, -1)
    out = _make_kernel(B, K)(pu_table, qi_table, users2d, items2d)
    return out.reshape(B, 1)
